# Initial kernel scaffold; baseline (speedup 1.0000x reference)
import jax
import jax.numpy as jnp
from jax.experimental import pallas as pl


def kernel(features, edge_index, W1, b1, W2, b2):
    raise NotImplementedError("write your pallas kernel here")


# speedup vs baseline: 7.7534x; 7.7534x over previous
import functools

import jax
import jax.numpy as jnp
from jax import lax
from jax.experimental import pallas as pl
from jax.experimental.pallas import tpu as pltpu
from jax.experimental.pallas import tpu_sc as plsc

NC = 2
NS = 16
NW = NC * NS
LANES = 16
CH = 128
DEGW = 16
DH = 64


def _sc_mesh():
    return plsc.VectorSubcoreMesh(core_axis_name="c", subcore_axis_name="s")


def _make_sc_degrees(R, nchunks):
    rows_per_tile = R // NS
    n_init = rows_per_tile // CH

    @functools.partial(
        pl.kernel,
        out_type=(
            jax.ShapeDtypeStruct((NC, R, DEGW), jnp.float32),
            jax.ShapeDtypeStruct((NC, R, DEGW), jnp.float32),
        ),
        mesh=_sc_mesh(),
        scratch_types=[
            pltpu.VMEM((nchunks, CH), jnp.int32),
            pltpu.VMEM((nchunks, CH), jnp.int32),
            pltpu.VMEM((CH, DEGW), jnp.float32),
            pltpu.VMEM_SHARED((R, DEGW), jnp.float32),
            pltpu.VMEM_SHARED((R, DEGW), jnp.float32),
        ],
        compiler_params=pltpu.CompilerParams(use_tc_tiling_on_sc=False),
    )
    def body(src_hbm, dst_hbm, osrc_hbm, odst_hbm,
             src_v, dst_v, ones_v, hsrc_sh, hdst_sh):
        c = lax.axis_index("c")
        s = lax.axis_index("s")
        wid = s * NC + c
        pltpu.sync_copy(src_hbm.at[wid], src_v)
        pltpu.sync_copy(dst_hbm.at[wid], dst_v)

        zero = jnp.zeros((LANES,), jnp.float32)
        one = jnp.ones((LANES,), jnp.float32)

        @pl.loop(0, CH)
        def _z(i):
            ones_v[i, :] = zero

        base = s * rows_per_tile
        for k in range(n_init):
            pltpu.sync_copy(ones_v, hsrc_sh.at[pl.ds(base + k * CH, CH)])
            pltpu.sync_copy(ones_v, hdst_sh.at[pl.ds(base + k * CH, CH)])

        @pl.loop(0, CH)
        def _o(i):
            ones_v[i, :] = one

        plsc.subcore_barrier()

        @pl.loop(0, nchunks)
        def _scat(j):
            pltpu.sync_copy(ones_v, hsrc_sh.at[src_v.at[j]], add=True)
            pltpu.sync_copy(ones_v, hdst_sh.at[dst_v.at[j]], add=True)

        plsc.subcore_barrier()
        sl = pl.ds(base, rows_per_tile)
        pltpu.sync_copy(hsrc_sh.at[sl], osrc_hbm.at[c, sl])
        pltpu.sync_copy(hdst_sh.at[sl], odst_hbm.at[c, sl])

    return body


def _make_sc_layer(R, nchunks):
    rows_per_tile = R // NS
    n_init = rows_per_tile // CH

    @functools.partial(
        pl.kernel,
        out_type=(
            jax.ShapeDtypeStruct((NC, R, DH), jnp.float32),
            jax.ShapeDtypeStruct((NC, R, DH), jnp.float32),
        ),
        mesh=_sc_mesh(),
        scratch_types=[
            pltpu.VMEM((nchunks, CH), jnp.int32),
            pltpu.VMEM((nchunks, CH), jnp.int32),
            pltpu.VMEM((2, CH, DH), jnp.float32),
            pltpu.VMEM_SHARED((R, DH), jnp.float32),
            pltpu.SemaphoreType.DMA,
        ],
        compiler_params=pltpu.CompilerParams(use_tc_tiling_on_sc=False),
    )
    def body(z0_hbm, z1_hbm, src_hbm, dst_hbm, out0_hbm, out1_hbm,
             src_v, dst_v, rows_v, agg_sh, sem):
        c = lax.axis_index("c")
        s = lax.axis_index("s")
        wid = s * NC + c
        pltpu.sync_copy(src_hbm.at[wid], src_v)
        pltpu.sync_copy(dst_hbm.at[wid], dst_v)

        zero = jnp.zeros((LANES,), jnp.float32)
        base = s * rows_per_tile
        sl = pl.ds(base, rows_per_tile)

        for z_hbm, out_hbm in ((z0_hbm, out0_hbm), (z1_hbm, out1_hbm)):
            @pl.loop(0, CH)
            def _z(i):
                for q in range(DH // LANES):
                    rows_v[0, i, pl.ds(q * LANES, LANES)] = zero

            for k in range(n_init):
                pltpu.sync_copy(rows_v.at[0], agg_sh.at[pl.ds(base + k * CH, CH)])
            plsc.subcore_barrier()

            pltpu.async_copy(z_hbm.at[src_v.at[0]], rows_v.at[0], sem)

            @pl.loop(0, nchunks, step=2)
            def _main(j):
                for b in range(2):
                    jb = j + b
                    pltpu.make_async_copy(
                        z_hbm.at[src_v.at[0]], rows_v.at[b], sem).wait()

                    @pl.when(jb + 1 < nchunks)
                    def _prefetch():
                        pltpu.async_copy(
                            z_hbm.at[src_v.at[jb + 1]], rows_v.at[1 - b], sem)

                    pltpu.sync_copy(rows_v.at[b], agg_sh.at[dst_v.at[jb]],
                                    add=True)

            plsc.subcore_barrier()
            pltpu.sync_copy(agg_sh.at[sl], out_hbm.at[c, sl])

    return body


def _tc_pre(dsrc_p, ddst_p, feat, W1):
    R, D = feat.shape

    def body(dsp, ddp, x, w, z0_out, z1_out, ns_out, nd_out):
        ds_ = dsp[0, :, 0:1] + dsp[1, :, 0:1]
        dd_ = ddp[0, :, 0:1] + ddp[1, :, 0:1]
        ns = lax.rsqrt(jnp.maximum(ds_, 1.0))
        nd = lax.rsqrt(jnp.maximum(dd_, 1.0))
        ns_out[...] = ns
        nd_out[...] = nd
        z = jnp.dot(x[...] * ns, w[...], preferred_element_type=jnp.float32)
        z0_out[...] = z[:, :DH]
        z1_out[...] = z[:, DH:]

    return pl.pallas_call(
        body,
        out_shape=(
            jax.ShapeDtypeStruct((R, DH), jnp.float32),
            jax.ShapeDtypeStruct((R, DH), jnp.float32),
            jax.ShapeDtypeStruct((R, 1), jnp.float32),
            jax.ShapeDtypeStruct((R, 1), jnp.float32),
        ),
    )(dsrc_p, ddst_p, feat, W1)


def _tc_mid(p0, p1, ns, nd, b1, W2):
    _, R, _ = p0.shape

    def body(p0_ref, p1_ref, ns_ref, nd_ref, b_ref, w_ref, z0_out, z1_out):
        h = jnp.concatenate(
            [p0_ref[0] + p0_ref[1], p1_ref[0] + p1_ref[1]], axis=1)
        h = jnp.maximum(h * nd_ref[...] + b_ref[...], 0.0)
        z = jnp.dot(h * ns_ref[...], w_ref[...],
                    preferred_element_type=jnp.float32)
        z0_out[...] = z[:, :DH]
        z1_out[...] = z[:, DH:]

    return pl.pallas_call(
        body,
        out_shape=(
            jax.ShapeDtypeStruct((R, DH), jnp.float32),
            jax.ShapeDtypeStruct((R, DH), jnp.float32),
        ),
    )(p0, p1, ns, nd, b1, W2)


def _tc_post(p0, p1, nd, b2):
    _, R, _ = p0.shape

    def body(p0_ref, p1_ref, nd_ref, b_ref, out):
        h = jnp.concatenate(
            [p0_ref[0] + p0_ref[1], p1_ref[0] + p1_ref[1]], axis=1)
        out[...] = h * nd_ref[...] + b_ref[...]

    return pl.pallas_call(
        body,
        out_shape=jax.ShapeDtypeStruct((R, 2 * DH), jnp.float32),
    )(p0, p1, nd, b2)


def kernel(features, edge_index, W1, b1, W2, b2):
    N, D = features.shape
    E = edge_index.shape[1]

    R = pl.cdiv(N + 1, NS * CH) * (NS * CH)
    epw = pl.cdiv(E, NW * 2 * CH) * (2 * CH)
    nchunks = epw // CH
    pad = NW * epw - E

    src = edge_index[0]
    dst = edge_index[1]
    trash = (N + jnp.arange(pad, dtype=jnp.int32) % (R - N)).astype(jnp.int32)
    srcp = jnp.concatenate([src, trash]).reshape(NW, nchunks, CH)
    dstp = jnp.concatenate([dst, trash]).reshape(NW, nchunks, CH)

    feat = jnp.pad(features, ((0, R - N), (0, 0)))

    sc_deg = _make_sc_degrees(R, nchunks)
    sc_layer = _make_sc_layer(R, nchunks)

    dsrc_p, ddst_p = sc_deg(srcp, dstp)
    z1a, z1b, ns, nd = _tc_pre(dsrc_p, ddst_p, feat, W1)
    p1a, p1b = sc_layer(z1a, z1b, srcp, dstp)
    z2a, z2b = _tc_mid(p1a, p1b, ns, nd, b1.reshape(1, D), W2)
    p2a, p2b = sc_layer(z2a, z2b, srcp, dstp)
    out = _tc_post(p2a, p2b, nd, b2.reshape(1, D))
    return out[:N]


# speedup vs baseline: 10.2394x; 1.3206x over previous
import functools

import jax
import jax.numpy as jnp
from jax import lax
from jax.experimental import pallas as pl
from jax.experimental.pallas import tpu as pltpu
from jax.experimental.pallas import tpu_sc as plsc

NC = 2
NS = 16
NW = NC * NS
LANES = 16
CH = 128
DEGW = 16
DH = 64


def _sc_mesh():
    return plsc.VectorSubcoreMesh(core_axis_name="c", subcore_axis_name="s")


def _make_sc_degrees(R, nchunks):
    rows_per_tile = R // NS
    n_init = rows_per_tile // CH

    @functools.partial(
        pl.kernel,
        out_type=(
            jax.ShapeDtypeStruct((NC, R, DEGW), jnp.float32),
            jax.ShapeDtypeStruct((NC, R, DEGW), jnp.float32),
        ),
        mesh=_sc_mesh(),
        scratch_types=[
            pltpu.VMEM((nchunks, CH), jnp.int32),
            pltpu.VMEM((nchunks, CH), jnp.int32),
            pltpu.VMEM((CH, DEGW), jnp.float32),
            pltpu.VMEM_SHARED((R, DEGW), jnp.float32),
            pltpu.VMEM_SHARED((R, DEGW), jnp.float32),
        ],
        compiler_params=pltpu.CompilerParams(use_tc_tiling_on_sc=False),
    )
    def body(src_hbm, dst_hbm, osrc_hbm, odst_hbm,
             src_v, dst_v, ones_v, hsrc_sh, hdst_sh):
        c = lax.axis_index("c")
        s = lax.axis_index("s")
        wid = s * NC + c
        pltpu.sync_copy(src_hbm.at[wid], src_v)
        pltpu.sync_copy(dst_hbm.at[wid], dst_v)

        zero = jnp.zeros((LANES,), jnp.float32)
        one = jnp.ones((LANES,), jnp.float32)

        @pl.loop(0, CH)
        def _z(i):
            ones_v[i, :] = zero

        base = s * rows_per_tile
        for k in range(n_init):
            pltpu.sync_copy(ones_v, hsrc_sh.at[pl.ds(base + k * CH, CH)])
            pltpu.sync_copy(ones_v, hdst_sh.at[pl.ds(base + k * CH, CH)])

        @pl.loop(0, CH)
        def _o(i):
            ones_v[i, :] = one

        plsc.subcore_barrier()

        @pl.loop(0, nchunks)
        def _scat(j):
            pltpu.sync_copy(ones_v, hsrc_sh.at[src_v.at[j]], add=True)
            pltpu.sync_copy(ones_v, hdst_sh.at[dst_v.at[j]], add=True)

        plsc.subcore_barrier()
        sl = pl.ds(base, rows_per_tile)
        pltpu.sync_copy(hsrc_sh.at[sl], osrc_hbm.at[c, sl])
        pltpu.sync_copy(hdst_sh.at[sl], odst_hbm.at[c, sl])

    return body


def _make_sc_layer(R, nchunks):
    rows_per_tile = R // NS
    n_init = rows_per_tile // CH

    @functools.partial(
        pl.kernel,
        out_type=(
            jax.ShapeDtypeStruct((NC, R, DH), jnp.float32),
            jax.ShapeDtypeStruct((NC, R, DH), jnp.float32),
        ),
        mesh=_sc_mesh(),
        scratch_types=[
            pltpu.VMEM((nchunks, CH), jnp.int32),
            pltpu.VMEM((nchunks, CH), jnp.int32),
            pltpu.VMEM((4, CH, DH), jnp.float32),
            pltpu.VMEM_SHARED((R, DH), jnp.float32),
            pltpu.SemaphoreType.DMA,
            pltpu.SemaphoreType.DMA,
        ],
        compiler_params=pltpu.CompilerParams(use_tc_tiling_on_sc=False),
    )
    def body(z0_hbm, z1_hbm, src_hbm, dst_hbm, out0_hbm, out1_hbm,
             src_v, dst_v, rows_v, agg_sh, sem, sem_s):
        c = lax.axis_index("c")
        s = lax.axis_index("s")
        wid = s * NC + c
        pltpu.sync_copy(src_hbm.at[wid], src_v)
        pltpu.sync_copy(dst_hbm.at[wid], dst_v)

        zero = jnp.zeros((LANES,), jnp.float32)
        base = s * rows_per_tile
        sl = pl.ds(base, rows_per_tile)

        for z_hbm, out_hbm in ((z0_hbm, out0_hbm), (z1_hbm, out1_hbm)):
            @pl.loop(0, CH)
            def _z(i):
                for q in range(DH // LANES):
                    rows_v[0, i, pl.ds(q * LANES, LANES)] = zero

            for k in range(n_init):
                pltpu.sync_copy(rows_v.at[0], agg_sh.at[pl.ds(base + k * CH, CH)])
            plsc.subcore_barrier()

            pltpu.async_copy(z_hbm.at[src_v.at[0]], rows_v.at[0], sem)
            pltpu.async_copy(z_hbm.at[src_v.at[1]], rows_v.at[1], sem)

            @pl.loop(0, nchunks, step=4)
            def _main(j):
                for b in range(4):
                    jb = j + b
                    pltpu.make_async_copy(
                        z_hbm.at[src_v.at[0]], rows_v.at[b], sem).wait()
                    pltpu.async_copy(rows_v.at[b], agg_sh.at[dst_v.at[jb]],
                                     sem_s, add=True)

                    @pl.when(jb + 2 < nchunks)
                    def _prefetch():
                        @pl.when(jb >= 2)
                        def _drain():
                            pltpu.make_async_copy(
                                z_hbm.at[src_v.at[0]],
                                rows_v.at[(b + 2) % 4], sem_s).wait()

                        pltpu.async_copy(
                            z_hbm.at[src_v.at[jb + 2]],
                            rows_v.at[(b + 2) % 4], sem)

            for _ in range(4):
                pltpu.make_async_copy(
                    z_hbm.at[src_v.at[0]], rows_v.at[0], sem_s).wait()
            plsc.subcore_barrier()
            pltpu.sync_copy(agg_sh.at[sl], out_hbm.at[c, sl])

    return body


def _tc_pre(dsrc_p, ddst_p, feat, W1):
    R, D = feat.shape

    def body(dsp, ddp, x, w, z0_out, z1_out, ns_out, nd_out):
        ds_ = dsp[0, :, 0:1] + dsp[1, :, 0:1]
        dd_ = ddp[0, :, 0:1] + ddp[1, :, 0:1]
        ns = lax.rsqrt(jnp.maximum(ds_, 1.0))
        nd = lax.rsqrt(jnp.maximum(dd_, 1.0))
        ns_out[...] = ns
        nd_out[...] = nd
        z = jnp.dot(x[...] * ns, w[...], preferred_element_type=jnp.float32)
        z0_out[...] = z[:, :DH]
        z1_out[...] = z[:, DH:]

    return pl.pallas_call(
        body,
        out_shape=(
            jax.ShapeDtypeStruct((R, DH), jnp.float32),
            jax.ShapeDtypeStruct((R, DH), jnp.float32),
            jax.ShapeDtypeStruct((R, 1), jnp.float32),
            jax.ShapeDtypeStruct((R, 1), jnp.float32),
        ),
    )(dsrc_p, ddst_p, feat, W1)


def _tc_mid(p0, p1, ns, nd, b1, W2):
    _, R, _ = p0.shape

    def body(p0_ref, p1_ref, ns_ref, nd_ref, b_ref, w_ref, z0_out, z1_out):
        h = jnp.concatenate(
            [p0_ref[0] + p0_ref[1], p1_ref[0] + p1_ref[1]], axis=1)
        h = jnp.maximum(h * nd_ref[...] + b_ref[...], 0.0)
        z = jnp.dot(h * ns_ref[...], w_ref[...],
                    preferred_element_type=jnp.float32)
        z0_out[...] = z[:, :DH]
        z1_out[...] = z[:, DH:]

    return pl.pallas_call(
        body,
        out_shape=(
            jax.ShapeDtypeStruct((R, DH), jnp.float32),
            jax.ShapeDtypeStruct((R, DH), jnp.float32),
        ),
    )(p0, p1, ns, nd, b1, W2)


def _tc_post(p0, p1, nd, b2):
    _, R, _ = p0.shape

    def body(p0_ref, p1_ref, nd_ref, b_ref, out):
        h = jnp.concatenate(
            [p0_ref[0] + p0_ref[1], p1_ref[0] + p1_ref[1]], axis=1)
        out[...] = h * nd_ref[...] + b_ref[...]

    return pl.pallas_call(
        body,
        out_shape=jax.ShapeDtypeStruct((R, 2 * DH), jnp.float32),
    )(p0, p1, nd, b2)


def kernel(features, edge_index, W1, b1, W2, b2):
    N, D = features.shape
    E = edge_index.shape[1]

    R = pl.cdiv(N + 1, NS * CH) * (NS * CH)
    epw = pl.cdiv(E, NW * 4 * CH) * (4 * CH)
    nchunks = epw // CH
    pad = NW * epw - E

    src = edge_index[0]
    dst = edge_index[1]
    trash = (N + jnp.arange(pad, dtype=jnp.int32) % (R - N)).astype(jnp.int32)
    srcp = jnp.concatenate([src, trash]).reshape(NW, nchunks, CH)
    dstp = jnp.concatenate([dst, trash]).reshape(NW, nchunks, CH)

    feat = jnp.pad(features, ((0, R - N), (0, 0)))

    sc_deg = _make_sc_degrees(R, nchunks)
    sc_layer = _make_sc_layer(R, nchunks)

    dsrc_p, ddst_p = sc_deg(srcp, dstp)
    z1a, z1b, ns, nd = _tc_pre(dsrc_p, ddst_p, feat, W1)
    p1a, p1b = sc_layer(z1a, z1b, srcp, dstp)
    z2a, z2b = _tc_mid(p1a, p1b, ns, nd, b1.reshape(1, D), W2)
    p2a, p2b = sc_layer(z2a, z2b, srcp, dstp)
    out = _tc_post(p2a, p2b, nd, b2.reshape(1, D))
    return out[:N]
